# baseline (speedup 1.0000x reference)
import functools

import jax
import jax.numpy as jnp
from jax import lax
from jax.experimental import pallas as pl
from jax.experimental.pallas import tpu as pltpu
from jax.experimental.pallas import tpu_sc as plsc

B = 16384
F = 32
L = 16
NC = 2
NS = 16
NW = NC * NS
BPW = B // NW
NCHUNK = 4
CHUNK = BPW // NCHUNK
NGROUP = BPW // L
RING = 2

_mesh = plsc.VectorSubcoreMesh(core_axis_name="c", subcore_axis_name="s")


@functools.partial(
    pl.kernel,
    out_type=jax.ShapeDtypeStruct((B,), jnp.float32),
    mesh=_mesh,
    compiler_params=pltpu.CompilerParams(
        needs_layout_passes=False, use_tc_tiling_on_sc=False),
)
def _sc_fused(user_hbm, item_hbm, ctx_hbm, params_hbm, utab_hbm, itab_hbm,
              out_hbm):
    wid = lax.axis_index("s") * NC + lax.axis_index("c")

    def body(uidx, iidx, ubuf, vbuf, ctxbuf, pbuf, obuf, sem):
        pltpu.sync_copy(user_hbm.at[wid], uidx)
        pltpu.sync_copy(item_hbm.at[wid], iidx)
        pltpu.sync_copy(ctx_hbm.at[0, wid], ctxbuf.at[0])
        pltpu.sync_copy(ctx_hbm.at[1, wid], ctxbuf.at[1])
        pltpu.sync_copy(params_hbm, pbuf)

        pend = []
        for f in range(F):
            for j in range(NCHUNK):
                sl = pl.ds(j * CHUNK, CHUNK)
                pend.append(pltpu.async_copy(
                    utab_hbm.at[f].at[uidx.at[j]], ubuf.at[f, sl], sem))
                pend.append(pltpu.async_copy(
                    itab_hbm.at[f].at[iidx.at[j]], vbuf.at[f, sl], sem))
            if f >= RING:
                for c in pend[:2 * NCHUNK]:
                    c.wait()
                pend = pend[2 * NCHUNK:]
        for c in pend:
            c.wait()

        g0 = pbuf[F * L:F * L + L]
        g1 = pbuf[F * L + L:F * L + 2 * L]
        base = pbuf[F * L + 2 * L:F * L + 3 * L]

        def g_body(g, carry):
            sl = pl.ds(g * L, L)
            acc = base + ctxbuf[0, sl] * g0 + ctxbuf[1, sl] * g1
            for f in range(F):
                w = pbuf[f * L:(f + 1) * L]
                acc = acc + ubuf[f, sl] * vbuf[f, sl] * w
            obuf[sl] = acc
            return carry

        lax.fori_loop(0, NGROUP, g_body, 0)

        pltpu.sync_copy(obuf, out_hbm.at[pl.ds(wid * BPW, BPW)])

    pl.run_scoped(
        body,
        pltpu.VMEM((NCHUNK, CHUNK), jnp.int32),
        pltpu.VMEM((NCHUNK, CHUNK), jnp.int32),
        pltpu.VMEM((F, BPW), jnp.float32),
        pltpu.VMEM((F, BPW), jnp.float32),
        pltpu.VMEM((2, BPW), jnp.float32),
        pltpu.VMEM((F * L + 3 * L,), jnp.float32),
        pltpu.VMEM((BPW,), jnp.float32),
        pltpu.SemaphoreType.DMA,
    )


def kernel(user, item, context, user_table, item_table, Wc, bc, Wo, bo):
    user3 = user.astype(jnp.int32).reshape(NW, NCHUNK, CHUNK)
    item3 = item.astype(jnp.int32).reshape(NW, NCHUNK, CHUNK)
    ctx3 = context.T.reshape(2, NW, BPW)
    utab_t = user_table.T
    itab_t = item_table.T
    wo = Wo.reshape(F)
    g = Wc @ wo
    base = bc @ wo + bo[0]
    params = jnp.concatenate(
        [jnp.broadcast_to(wo[:, None], (F, L)).reshape(F * L),
         jnp.full((L,), g[0], jnp.float32),
         jnp.full((L,), g[1], jnp.float32),
         jnp.full((L,), base, jnp.float32)]
    )
    return _sc_fused(user3, item3, ctx3, params, utab_t, itab_t)


# speedup vs baseline: 5.6541x; 5.6541x over previous
import functools

import jax
import jax.numpy as jnp
from jax import lax
from jax.experimental import pallas as pl
from jax.experimental.pallas import tpu as pltpu
from jax.experimental.pallas import tpu_sc as plsc

B = 16384
F = 32
L = 16
NC = 2
NS = 16
NW = NC * NS
BPW = B // NW
NCHUNK = 4
CHUNK = BPW // NCHUNK
NGROUP = BPW // L

_mesh = plsc.VectorSubcoreMesh(core_axis_name="c", subcore_axis_name="s")


@functools.partial(
    pl.kernel,
    out_type=jax.ShapeDtypeStruct((B,), jnp.float32),
    mesh=_mesh,
    compiler_params=pltpu.CompilerParams(
        needs_layout_passes=False, use_tc_tiling_on_sc=False),
)
def _sc_fused(user_hbm, item_hbm, ctx_hbm, params_hbm, utab_hbm, itab_hbm,
              out_hbm):
    wid = lax.axis_index("s") * NC + lax.axis_index("c")

    def body(uidx, iidx, ubuf, vbuf, ctxbuf, pbuf, obuf, sem):
        pltpu.sync_copy(user_hbm.at[wid], uidx)
        pltpu.sync_copy(item_hbm.at[wid], iidx)
        pltpu.sync_copy(ctx_hbm.at[0, wid], ctxbuf.at[0])
        pltpu.sync_copy(ctx_hbm.at[1, wid], ctxbuf.at[1])
        pltpu.sync_copy(params_hbm, pbuf)

        copies = []
        for j in range(NCHUNK):
            dst = ubuf.at[pl.ds(j * CHUNK, CHUNK), :]
            copies.append(pltpu.async_copy(utab_hbm.at[uidx.at[j]], dst, sem))
        for j in range(NCHUNK):
            dst = vbuf.at[pl.ds(j * CHUNK, CHUNK), :]
            copies.append(pltpu.async_copy(itab_hbm.at[iidx.at[j]], dst, sem))
        for c in copies:
            c.wait()

        g0 = pbuf[F * L:F * L + L]
        g1 = pbuf[F * L + L:F * L + 2 * L]
        base = pbuf[F * L + 2 * L:F * L + 3 * L]

        lanes = lax.iota(jnp.int32, L)

        def g_body(g, carry):
            off = g * L
            sl = pl.ds(off, L)
            rows = off + lanes
            acc = base + ctxbuf[0, sl] * g0 + ctxbuf[1, sl] * g1
            for f in range(F):
                fv = jnp.full((L,), f, jnp.int32)
                w = pbuf[f * L:(f + 1) * L]
                ug = plsc.load_gather(ubuf, [rows, fv])
                vg = plsc.load_gather(vbuf, [rows, fv])
                acc = acc + ug * vg * w
            obuf[sl] = acc
            return carry

        lax.fori_loop(0, NGROUP, g_body, 0)

        pltpu.sync_copy(obuf, out_hbm.at[pl.ds(wid * BPW, BPW)])

    pl.run_scoped(
        body,
        pltpu.VMEM((NCHUNK, CHUNK), jnp.int32),
        pltpu.VMEM((NCHUNK, CHUNK), jnp.int32),
        pltpu.VMEM((BPW, F), jnp.float32),
        pltpu.VMEM((BPW, F), jnp.float32),
        pltpu.VMEM((2, BPW), jnp.float32),
        pltpu.VMEM((F * L + 3 * L,), jnp.float32),
        pltpu.VMEM((BPW,), jnp.float32),
        pltpu.SemaphoreType.DMA,
    )


def kernel(user, item, context, user_table, item_table, Wc, bc, Wo, bo):
    user3 = user.astype(jnp.int32).reshape(NW, NCHUNK, CHUNK)
    item3 = item.astype(jnp.int32).reshape(NW, NCHUNK, CHUNK)
    ctx3 = context.T.reshape(2, NW, BPW)
    wo = Wo.reshape(F)
    g = Wc @ wo
    base = bc @ wo + bo[0]
    params = jnp.concatenate(
        [jnp.broadcast_to(wo[:, None], (F, L)).reshape(F * L),
         jnp.full((L,), g[0], jnp.float32),
         jnp.full((L,), g[1], jnp.float32),
         jnp.full((L,), base, jnp.float32)]
    )
    return _sc_fused(user3, item3, ctx3, params, user_table, item_table)
